# baseline (speedup 1.0000x reference)
import jax
import jax.numpy as jnp
from jax.experimental import pallas as pl
from jax.experimental.pallas import tpu as pltpu

_R = 512
_W = 128


def _block_kernel(cond_ref, z_ref, out_ref):
    K = z_ref.shape[1]
    c = cond_ref[...][:, 0]
    neg_inf = jnp.float32(-jnp.inf)
    strips = [(s0, min(_W, K - s0)) for s0 in range(0, K, _W)]

    acc_m = jnp.full((_R,), neg_inf, jnp.float32)
    acc_t = jnp.full((_R,), neg_inf, jnp.float32)
    for s0, w in strips:
        zs = z_ref[:, s0:s0 + w]
        cols = s0 + jax.lax.broadcasted_iota(jnp.int32, (_R, w), 1)
        ist = cols == c[:, None]
        acc_m = jnp.maximum(acc_m, jnp.max(jnp.where(ist, neg_inf, zs), axis=1))
        acc_t = jnp.maximum(acc_t, jnp.max(jnp.where(ist, zs, neg_inf), axis=1))

    m = jnp.maximum(acc_m, jnp.float32(0.0))
    v = jnp.logaddexp(acc_t, m)
    krow = c == K
    any_k = jnp.any(krow)

    @pl.when(jnp.logical_not(any_k))
    def _():
        for s0, w in strips:
            zs = z_ref[:, s0:s0 + w]
            cols = s0 + jax.lax.broadcasted_iota(jnp.int32, (_R, w), 1)
            ist = cols == c[:, None]
            out_ref[:, s0:s0 + w] = jnp.where(ist, v[:, None], zs)

    @pl.when(any_k)
    def _():
        for s0, w in strips:
            zs = z_ref[:, s0:s0 + w]
            cols = s0 + jax.lax.broadcasted_iota(jnp.int32, (_R, w), 1)
            ist = cols == c[:, None]
            res = jnp.where(ist, v[:, None], zs)
            sp = -jax.nn.softplus(-zs)
            out_ref[:, s0:s0 + w] = jnp.where(krow[:, None], sp, res)


def kernel(z, cond):
    N, K = z.shape
    cond2 = cond.reshape(N, 1)
    grid = (N // _R,)
    return pl.pallas_call(
        _block_kernel,
        grid=grid,
        in_specs=[
            pl.BlockSpec((_R, 1), lambda i: (i, 0)),
            pl.BlockSpec((_R, K), lambda i: (i, 0)),
        ],
        out_specs=pl.BlockSpec((_R, K), lambda i: (i, 0)),
        out_shape=jax.ShapeDtypeStruct((N, K), z.dtype),
        compiler_params=pltpu.CompilerParams(
            dimension_semantics=("arbitrary",),
        ),
    )(cond2, z)
